# Initial kernel scaffold; baseline (speedup 1.0000x reference)
import jax
import jax.numpy as jnp
from jax.experimental import pallas as pl


def kernel(byte_seq, emb, phi_w, phi_b, amp_w, amp_b, router_w, router_b, w1, b1, w2, b2, head_w, head_b):
    raise NotImplementedError("write your pallas kernel here")


# speedup vs baseline: 24.8814x; 24.8814x over previous
import math
import functools

import jax
import jax.numpy as jnp
from jax import lax
from jax.experimental import pallas as pl
from jax.experimental.pallas import tpu as pltpu

D_MODEL = 768
N_EXPERTS = 8
D_FF = 4 * D_MODEL
VOCAB = 256
T_SEQ = 2048
TOL = 0.15


T_BLK = 512


def _pre_body(byte_ref, emb_ref, phiw_ref, phib_ref, ampw_ref, ampb_ref,
              rw_ref, rb_ref,
              y_ref, ybf_ref, gates_ref, state_ref,
              a_ref, s_ref, g_ref, st_ref):
    i = pl.program_id(0)

    b = byte_ref[:, :]
    oh = (b == lax.broadcasted_iota(jnp.int32, (T_BLK, VOCAB), 1)).astype(jnp.float32)
    x = jnp.dot(oh, emb_ref[:, :], preferred_element_type=jnp.float32)

    raw = jnp.dot(x, phiw_ref[:, :],
                  preferred_element_type=jnp.float32) + phib_ref[:, :]
    ang = jnp.tanh(raw) * math.pi
    step = math.pi / 9.0
    harm = jnp.round(ang * (1.0 / step)) * step
    ang = jnp.where(jnp.abs(ang - harm) < TOL, harm, ang)
    sn = jnp.sin(ang)
    a_ref[:, :] = jnp.cos(ang) + sn
    s_ref[:, :] = sn
    g_ref[:, :] = jax.nn.sigmoid(
        jnp.dot(x, ampw_ref[:, :], preferred_element_type=jnp.float32) + ampb_ref[:, :])

    @pl.when(i == 0)
    def _init_state():
        st_ref[:, :] = jnp.zeros((1, D_MODEL), jnp.float32)

    def body(t, st):
        new = jnp.clip(a_ref[pl.ds(t, 1), :] * st - s_ref[pl.ds(t, 1), :],
                       -1.0, 1.0)
        y_ref[pl.ds(t, 1), :] = g_ref[pl.ds(t, 1), :] * new
        return new

    st_ref[:, :] = lax.fori_loop(0, T_BLK, body, st_ref[:, :])

    @pl.when(i == pl.num_programs(0) - 1)
    def _emit_state():
        state_ref[:, :] = st_ref[:, :]

    y = y_ref[:, :]
    ybf_ref[:, :] = y.astype(jnp.bfloat16)

    lg = jnp.dot(y, rw_ref[:, :],
                 preferred_element_type=jnp.float32) + rb_ref[:, :]
    m1 = jnp.max(lg, axis=1, keepdims=True)
    masked = jnp.where(lg >= m1, -jnp.inf, lg)
    m2 = jnp.max(masked, axis=1, keepdims=True)
    g1 = 1.0 / (1.0 + jnp.exp(m2 - m1))
    gates_ref[:, :] = jnp.where(lg >= m1, g1,
                                jnp.where(lg >= m2, 1.0 - g1, 0.0))


def _moe_body(ybf_ref, gates_ref, w1_ref, b1_ref, w2_ref, b2_ref, acc_ref):
    e = pl.program_id(0)
    f = pl.program_id(1)

    @pl.when(jnp.logical_and(e == 0, f == 0))
    def _init():
        acc_ref[:, :] = jnp.zeros_like(acc_ref)

    xb = ybf_ref[:, :]
    w1b = w1_ref[0, :, :].astype(jnp.bfloat16)
    h = jnp.dot(xb, w1b, preferred_element_type=jnp.float32) + b1_ref[0, :, :]
    h = jax.nn.gelu(h)
    w2b = w2_ref[0, :, :].astype(jnp.bfloat16)
    part = jnp.dot(h.astype(jnp.bfloat16), w2b,
                   preferred_element_type=jnp.float32)

    lane = lax.broadcasted_iota(jnp.int32, (T_SEQ, N_EXPERTS), 1)
    gate = jnp.sum(jnp.where(lane == e, gates_ref[:, :], 0.0), axis=1,
                   keepdims=True)

    @pl.when(f == 0)
    def _bias():
        acc_ref[:, :] += gate * (part + b2_ref[0, :, :])

    @pl.when(f != 0)
    def _nobias():
        acc_ref[:, :] += gate * part


def _head_body(x_ref, hw_ref, hb_ref, out_ref):
    xb = x_ref[:, :].astype(jnp.bfloat16)
    wb = hw_ref[:, :].astype(jnp.bfloat16)
    out_ref[:, :] = (jnp.dot(xb, wb, preferred_element_type=jnp.float32)
                     + hb_ref[:, :])


def kernel(byte_seq, emb, phi_w, phi_b, amp_w, amp_b, router_w, router_b,
           w1, b1, w2, b2, head_w, head_b):
    byte_col = byte_seq.reshape(T_SEQ, 1).astype(jnp.int32)

    f32 = jnp.float32
    n_tb = T_SEQ // T_BLK
    y, ybf, gates, state = pl.pallas_call(
        _pre_body,
        grid=(n_tb,),
        in_specs=[
            pl.BlockSpec((T_BLK, 1), lambda i: (i, 0)),
            pl.BlockSpec((VOCAB, D_MODEL), lambda i: (0, 0)),
            pl.BlockSpec((D_MODEL, D_MODEL), lambda i: (0, 0)),
            pl.BlockSpec((1, D_MODEL), lambda i: (0, 0)),
            pl.BlockSpec((D_MODEL, D_MODEL), lambda i: (0, 0)),
            pl.BlockSpec((1, D_MODEL), lambda i: (0, 0)),
            pl.BlockSpec((D_MODEL, N_EXPERTS), lambda i: (0, 0)),
            pl.BlockSpec((1, N_EXPERTS), lambda i: (0, 0)),
        ],
        out_specs=[
            pl.BlockSpec((T_BLK, D_MODEL), lambda i: (i, 0)),
            pl.BlockSpec((T_BLK, D_MODEL), lambda i: (i, 0)),
            pl.BlockSpec((T_BLK, N_EXPERTS), lambda i: (i, 0)),
            pl.BlockSpec((1, D_MODEL), lambda i: (0, 0)),
        ],
        out_shape=[
            jax.ShapeDtypeStruct((T_SEQ, D_MODEL), f32),
            jax.ShapeDtypeStruct((T_SEQ, D_MODEL), jnp.bfloat16),
            jax.ShapeDtypeStruct((T_SEQ, N_EXPERTS), f32),
            jax.ShapeDtypeStruct((1, D_MODEL), f32),
        ],
        scratch_shapes=[
            pltpu.VMEM((T_BLK, D_MODEL), f32),
            pltpu.VMEM((T_BLK, D_MODEL), f32),
            pltpu.VMEM((T_BLK, D_MODEL), f32),
            pltpu.VMEM((1, D_MODEL), f32),
        ],
        compiler_params=pltpu.CompilerParams(
            dimension_semantics=("arbitrary",),
            vmem_limit_bytes=100 * 2**20,
        ),
    )(byte_col, emb, phi_w, phi_b.reshape(1, D_MODEL),
      amp_w, amp_b.reshape(1, D_MODEL),
      router_w, router_b.reshape(1, N_EXPERTS))

    ff_blk = 768
    n_ff = D_FF // ff_blk
    moe_out = pl.pallas_call(
        _moe_body,
        grid=(N_EXPERTS, n_ff),
        in_specs=[
            pl.BlockSpec((T_SEQ, D_MODEL), lambda e, f: (0, 0)),
            pl.BlockSpec((T_SEQ, N_EXPERTS), lambda e, f: (0, 0)),
            pl.BlockSpec((1, D_MODEL, ff_blk), lambda e, f: (e, 0, f)),
            pl.BlockSpec((1, 1, ff_blk), lambda e, f: (e, 0, f)),
            pl.BlockSpec((1, ff_blk, D_MODEL), lambda e, f: (e, f, 0)),
            pl.BlockSpec((1, 1, D_MODEL), lambda e, f: (e, 0, 0)),
        ],
        out_specs=pl.BlockSpec((T_SEQ, D_MODEL), lambda e, f: (0, 0)),
        out_shape=jax.ShapeDtypeStruct((T_SEQ, D_MODEL), f32),
        compiler_params=pltpu.CompilerParams(
            dimension_semantics=("arbitrary", "arbitrary"),
            vmem_limit_bytes=100 * 2**20,
        ),
    )(ybf, gates, w1, b1.reshape(N_EXPERTS, 1, D_FF), w2,
      b2.reshape(N_EXPERTS, 1, D_MODEL))

    logits = pl.pallas_call(
        _head_body,
        out_shape=jax.ShapeDtypeStruct((T_SEQ, VOCAB), f32),
        compiler_params=pltpu.CompilerParams(vmem_limit_bytes=100 * 2**20),
    )(moe_out, head_w, head_b.reshape(1, VOCAB))

    entropy_loss = jnp.zeros((), f32)
    return logits.reshape(1, T_SEQ, VOCAB), state, entropy_loss
